# baseline (speedup 1.0000x reference)
import functools

import jax
import jax.numpy as jnp
from jax import lax
from jax.experimental import pallas as pl
from jax.experimental.pallas import tpu as pltpu
from jax.experimental.pallas import tpu_sc as plsc

_NUM_WORKERS = 32
_LANES = 16


def _tc_logits(x, weight, chunk, tc_blk, tok_offset, tok_count):
    _, h = x.shape
    e = weight.shape[0]
    n_chunks = tok_count // chunk
    assert tc_blk % chunk == 0 and tok_offset % tc_blk == 0
    chunks_per_blk = tc_blk // chunk
    blk_offset = tok_offset // tc_blk

    del tc_blk
    nbuf = 8
    assert n_chunks % nbuf == 0

    def body(x_hbm, w_ref, out_ref, buf, sems):
        def start(slot, idx):
            pltpu.make_async_copy(
                x_hbm.at[pl.ds(idx * chunk + tok_offset, chunk), :],
                buf.at[slot],
                sems.at[slot],
            ).start()

        for b in range(nbuf):
            start(b, b)

        def outer(g, carry):
            for b in range(nbuf):
                idx = g * nbuf + b
                pltpu.make_async_copy(
                    x_hbm.at[pl.ds(0, chunk), :], buf.at[b], sems.at[b]
                ).wait()
                logits = lax.dot_general(
                    w_ref[...],
                    buf[b],
                    dimension_numbers=(((1,), (1,)), ((), ())),
                    preferred_element_type=jnp.float32,
                )

                @pl.when(idx + nbuf < n_chunks)
                def _():
                    start(b, idx + nbuf)

                out_ref[idx] = logits
            return carry

        lax.fori_loop(0, n_chunks // nbuf, outer, jnp.int32(0))

    return pl.pallas_call(
        body,
        in_specs=[
            pl.BlockSpec(memory_space=pltpu.HBM),
            pl.BlockSpec(memory_space=pltpu.VMEM),
        ],
        out_specs=pl.BlockSpec(memory_space=pltpu.VMEM),
        out_shape=jax.ShapeDtypeStruct((n_chunks, e, chunk), jnp.float32),
        scratch_shapes=[
            pltpu.VMEM((nbuf, chunk, h), jnp.float32),
            pltpu.SemaphoreType.DMA((nbuf,)),
        ],
    )(x, weight)


@functools.lru_cache(maxsize=None)
def _make_sc_route(t_tokens, e):
    chunk = t_tokens // _NUM_WORKERS
    steps = chunk // _LANES
    mesh = plsc.VectorSubcoreMesh(core_axis_name="c", subcore_axis_name="s")

    @functools.partial(
        pl.kernel,
        mesh=mesh,
        out_type=(
            jax.ShapeDtypeStruct((t_tokens,), jnp.int32),
            jax.ShapeDtypeStruct((t_tokens,), jnp.int32),
            jax.ShapeDtypeStruct((t_tokens,), jnp.float32),
            jax.ShapeDtypeStruct((t_tokens,), jnp.float32),
        ),
        scratch_types=[
            pltpu.VMEM((e, chunk), jnp.float32),
            pltpu.VMEM((chunk,), jnp.int32),
            pltpu.VMEM((chunk,), jnp.int32),
            pltpu.VMEM((chunk,), jnp.float32),
            pltpu.VMEM((chunk,), jnp.float32),
        ],
    )
    def route(logits_hbm, i1_hbm, i2_hbm, w1_hbm, w2_hbm, lv, iv1, iv2, wv1, wv2):
        num_cores = 2
        wid = lax.axis_index("s") * num_cores + lax.axis_index("c")
        base = wid * chunk
        pltpu.sync_copy(logits_hbm.at[wid], lv)

        def body(i, carry):
            t = i * _LANES
            scores = [lv[ex, pl.ds(t, _LANES)] for ex in range(e)]
            best = scores[0]
            bidx = jnp.zeros((_LANES,), jnp.int32)
            for ex in range(1, e):
                exv = jnp.full((_LANES,), ex, jnp.int32)
                m = scores[ex] > best
                best = jnp.where(m, scores[ex], best)
                bidx = jnp.where(m, exv, bidx)
            sec = jnp.full((_LANES,), -jnp.inf, jnp.float32)
            sidx = jnp.zeros((_LANES,), jnp.int32)
            for ex in range(e):
                exv = jnp.full((_LANES,), ex, jnp.int32)
                m = (scores[ex] > sec) & (bidx != exv)
                sec = jnp.where(m, scores[ex], sec)
                sidx = jnp.where(m, exv, sidx)
            w1 = 1.0 / (1.0 + jnp.exp(sec - best))
            w2 = 1.0 - w1
            iv1[pl.ds(t, _LANES)] = bidx
            iv2[pl.ds(t, _LANES)] = sidx
            wv1[pl.ds(t, _LANES)] = w1
            wv2[pl.ds(t, _LANES)] = w2
            return carry

        lax.fori_loop(0, steps, body, jnp.int32(0))
        pltpu.sync_copy(iv1, i1_hbm.at[pl.ds(base, chunk)])
        pltpu.sync_copy(iv2, i2_hbm.at[pl.ds(base, chunk)])
        pltpu.sync_copy(wv1, w1_hbm.at[pl.ds(base, chunk)])
        pltpu.sync_copy(wv2, w2_hbm.at[pl.ds(base, chunk)])

    return route


_NUM_SLICES = 1


def kernel(hidden_states, weight):
    b, s, h = hidden_states.shape
    t_tokens = b * s
    e = weight.shape[0]
    x = hidden_states.reshape(t_tokens, h).astype(jnp.float32)
    w = weight.astype(jnp.float32)
    t_slice = t_tokens // _NUM_SLICES
    chunk = t_slice // _NUM_WORKERS
    sc_route = _make_sc_route(t_slice, e)
    parts = []
    for si in range(_NUM_SLICES):
        logits = _tc_logits(
            x, w, chunk, tc_blk=2048, tok_offset=si * t_slice, tok_count=t_slice
        )
        parts.append(sc_route(logits))
    i1 = jnp.concatenate([p[0] for p in parts])
    i2 = jnp.concatenate([p[1] for p in parts])
    w1 = jnp.concatenate([p[2] for p in parts])
    w2 = jnp.concatenate([p[3] for p in parts])
    topk_idx = jnp.stack([i1, i2], axis=1)
    topk_weight = jnp.stack([w1, w2], axis=1)
    return topk_idx, topk_weight


# speedup vs baseline: 1.0464x; 1.0464x over previous
import functools

import jax
import jax.numpy as jnp
from jax import lax
from jax.experimental import pallas as pl
from jax.experimental.pallas import tpu as pltpu
from jax.experimental.pallas import tpu_sc as plsc

_NUM_WORKERS = 32
_LANES = 16


def _tc_logits(x, weight, chunk, tc_blk, tok_offset, tok_count):
    _, h = x.shape
    e = weight.shape[0]
    n_chunks = tok_count // chunk
    assert tc_blk % chunk == 0 and tok_offset % tc_blk == 0
    chunks_per_blk = tc_blk // chunk
    blk_offset = tok_offset // tc_blk

    def body(w_ref, x_ref, out_ref):
        for c in range(chunks_per_blk):
            out_ref[c] = lax.dot_general(
                w_ref[...],
                x_ref[pl.ds(c * chunk, chunk), :],
                dimension_numbers=(((1,), (1,)), ((), ())),
                preferred_element_type=jnp.float32,
            )

    return pl.pallas_call(
        body,
        grid=(tok_count // tc_blk,),
        in_specs=[
            pl.BlockSpec((e, h), lambda i: (0, 0)),
            pl.BlockSpec((tc_blk, h), lambda i: (i + blk_offset, 0)),
        ],
        out_specs=pl.BlockSpec((chunks_per_blk, e, chunk), lambda i: (i, 0, 0)),
        out_shape=jax.ShapeDtypeStruct((n_chunks, e, chunk), jnp.float32),
    )(weight, x)


@functools.lru_cache(maxsize=None)
def _make_sc_route(t_tokens, e):
    chunk = t_tokens // _NUM_WORKERS
    steps = chunk // _LANES
    mesh = plsc.VectorSubcoreMesh(core_axis_name="c", subcore_axis_name="s")

    @functools.partial(
        pl.kernel,
        mesh=mesh,
        out_type=(
            jax.ShapeDtypeStruct((t_tokens,), jnp.int32),
            jax.ShapeDtypeStruct((t_tokens,), jnp.int32),
            jax.ShapeDtypeStruct((t_tokens,), jnp.float32),
            jax.ShapeDtypeStruct((t_tokens,), jnp.float32),
        ),
        scratch_types=[
            pltpu.VMEM((e, chunk), jnp.float32),
            pltpu.VMEM((chunk,), jnp.int32),
            pltpu.VMEM((chunk,), jnp.int32),
            pltpu.VMEM((chunk,), jnp.float32),
            pltpu.VMEM((chunk,), jnp.float32),
        ],
    )
    def route(logits_hbm, i1_hbm, i2_hbm, w1_hbm, w2_hbm, lv, iv1, iv2, wv1, wv2):
        num_cores = 2
        wid = lax.axis_index("s") * num_cores + lax.axis_index("c")
        base = wid * chunk
        pltpu.sync_copy(logits_hbm.at[wid], lv)

        def body(i, carry):
            t = i * _LANES
            scores = [lv[ex, pl.ds(t, _LANES)] for ex in range(e)]
            best = scores[0]
            bidx = jnp.zeros((_LANES,), jnp.int32)
            for ex in range(1, e):
                exv = jnp.full((_LANES,), ex, jnp.int32)
                m = scores[ex] > best
                best = jnp.where(m, scores[ex], best)
                bidx = jnp.where(m, exv, bidx)
            sec = jnp.full((_LANES,), -jnp.inf, jnp.float32)
            sidx = jnp.zeros((_LANES,), jnp.int32)
            for ex in range(e):
                exv = jnp.full((_LANES,), ex, jnp.int32)
                m = (scores[ex] > sec) & (bidx != exv)
                sec = jnp.where(m, scores[ex], sec)
                sidx = jnp.where(m, exv, sidx)
            w1 = 1.0 / (1.0 + jnp.exp(sec - best))
            w2 = 1.0 - w1
            iv1[pl.ds(t, _LANES)] = bidx
            iv2[pl.ds(t, _LANES)] = sidx
            wv1[pl.ds(t, _LANES)] = w1
            wv2[pl.ds(t, _LANES)] = w2
            return carry

        lax.fori_loop(0, steps, body, jnp.int32(0))
        pltpu.sync_copy(iv1, i1_hbm.at[pl.ds(base, chunk)])
        pltpu.sync_copy(iv2, i2_hbm.at[pl.ds(base, chunk)])
        pltpu.sync_copy(wv1, w1_hbm.at[pl.ds(base, chunk)])
        pltpu.sync_copy(wv2, w2_hbm.at[pl.ds(base, chunk)])

    return route


_NUM_SLICES = 1


def kernel(hidden_states, weight):
    b, s, h = hidden_states.shape
    t_tokens = b * s
    e = weight.shape[0]
    x = hidden_states.reshape(t_tokens, h).astype(jnp.float32)
    w = weight.astype(jnp.float32)
    t_slice = t_tokens // _NUM_SLICES
    chunk = t_slice // _NUM_WORKERS
    sc_route = _make_sc_route(t_slice, e)
    parts = []
    for si in range(_NUM_SLICES):
        logits = _tc_logits(
            x, w, chunk, tc_blk=2048, tok_offset=si * t_slice, tok_count=t_slice
        )
        parts.append(sc_route(logits))
    i1 = jnp.concatenate([p[0] for p in parts])
    i2 = jnp.concatenate([p[1] for p in parts])
    w1 = jnp.concatenate([p[2] for p in parts])
    w2 = jnp.concatenate([p[3] for p in parts])
    topk_idx = jnp.stack([i1, i2], axis=1)
    topk_weight = jnp.stack([w1, w2], axis=1)
    return topk_idx, topk_weight


# speedup vs baseline: 1.0556x; 1.0088x over previous
import functools

import jax
import jax.numpy as jnp
from jax import lax
from jax.experimental import pallas as pl
from jax.experimental.pallas import tpu as pltpu
from jax.experimental.pallas import tpu_sc as plsc

_NUM_WORKERS = 32
_LANES = 16


def _tc_logits(x, weight, chunk, tc_blk, tok_offset, tok_count):
    _, h = x.shape
    e = weight.shape[0]
    n_chunks = tok_count // chunk
    assert tc_blk % chunk == 0 and tok_offset % tc_blk == 0
    chunks_per_blk = tc_blk // chunk
    blk_offset = tok_offset // tc_blk

    def body(w_ref, x_ref, out_ref):
        for c in range(chunks_per_blk):
            out_ref[c] = lax.dot_general(
                w_ref[...],
                x_ref[pl.ds(c * chunk, chunk), :],
                dimension_numbers=(((1,), (1,)), ((), ())),
                preferred_element_type=jnp.float32,
            )

    return pl.pallas_call(
        body,
        grid=(tok_count // tc_blk,),
        in_specs=[
            pl.BlockSpec((e, h), lambda i: (0, 0)),
            pl.BlockSpec((tc_blk, h), lambda i: (i + blk_offset, 0)),
        ],
        out_specs=pl.BlockSpec((chunks_per_blk, e, chunk), lambda i: (i, 0, 0)),
        out_shape=jax.ShapeDtypeStruct((n_chunks, e, chunk), jnp.float32),
    )(weight, x)


@functools.lru_cache(maxsize=None)
def _make_sc_route(t_tokens, e):
    chunk = t_tokens // _NUM_WORKERS
    steps = chunk // _LANES
    mesh = plsc.VectorSubcoreMesh(core_axis_name="c", subcore_axis_name="s")

    @functools.partial(
        pl.kernel,
        mesh=mesh,
        out_type=(
            jax.ShapeDtypeStruct((t_tokens,), jnp.int32),
            jax.ShapeDtypeStruct((t_tokens,), jnp.int32),
            jax.ShapeDtypeStruct((t_tokens,), jnp.float32),
            jax.ShapeDtypeStruct((t_tokens,), jnp.float32),
        ),
        scratch_types=[
            pltpu.VMEM((e, chunk), jnp.float32),
            pltpu.VMEM((chunk,), jnp.int32),
            pltpu.VMEM((chunk,), jnp.int32),
            pltpu.VMEM((chunk,), jnp.float32),
            pltpu.VMEM((chunk,), jnp.float32),
        ],
    )
    def route(logits_hbm, i1_hbm, i2_hbm, w1_hbm, w2_hbm, lv, iv1, iv2, wv1, wv2):
        num_cores = 2
        wid = lax.axis_index("s") * num_cores + lax.axis_index("c")
        base = wid * chunk
        pltpu.sync_copy(logits_hbm.at[wid], lv)

        def body(i):
            t = i * _LANES
            scores = [lv[ex, pl.ds(t, _LANES)] for ex in range(e)]
            best = scores[0]
            bidx = jnp.zeros((_LANES,), jnp.int32)
            for ex in range(1, e):
                exv = jnp.full((_LANES,), ex, jnp.int32)
                m = scores[ex] > best
                best = jnp.where(m, scores[ex], best)
                bidx = jnp.where(m, exv, bidx)
            sec = jnp.full((_LANES,), -jnp.inf, jnp.float32)
            sidx = jnp.zeros((_LANES,), jnp.int32)
            for ex in range(e):
                exv = jnp.full((_LANES,), ex, jnp.int32)
                m = (scores[ex] > sec) & (bidx != exv)
                sec = jnp.where(m, scores[ex], sec)
                sidx = jnp.where(m, exv, sidx)
            w1 = 1.0 / (1.0 + jnp.exp(sec - best))
            w2 = 1.0 - w1
            iv1[pl.ds(t, _LANES)] = bidx
            iv2[pl.ds(t, _LANES)] = sidx
            wv1[pl.ds(t, _LANES)] = w1
            wv2[pl.ds(t, _LANES)] = w2

        for i in range(steps):
            body(i)
        pltpu.sync_copy(iv1, i1_hbm.at[pl.ds(base, chunk)])
        pltpu.sync_copy(iv2, i2_hbm.at[pl.ds(base, chunk)])
        pltpu.sync_copy(wv1, w1_hbm.at[pl.ds(base, chunk)])
        pltpu.sync_copy(wv2, w2_hbm.at[pl.ds(base, chunk)])

    return route


_NUM_SLICES = 1


def kernel(hidden_states, weight):
    b, s, h = hidden_states.shape
    t_tokens = b * s
    e = weight.shape[0]
    x = hidden_states.reshape(t_tokens, h).astype(jnp.float32)
    w = weight.astype(jnp.float32)
    t_slice = t_tokens // _NUM_SLICES
    chunk = t_slice // _NUM_WORKERS
    sc_route = _make_sc_route(t_slice, e)
    parts = []
    for si in range(_NUM_SLICES):
        logits = _tc_logits(
            x, w, chunk, tc_blk=2048, tok_offset=si * t_slice, tok_count=t_slice
        )
        parts.append(sc_route(logits))
    i1 = jnp.concatenate([p[0] for p in parts])
    i2 = jnp.concatenate([p[1] for p in parts])
    w1 = jnp.concatenate([p[2] for p in parts])
    w2 = jnp.concatenate([p[3] for p in parts])
    topk_idx = jnp.stack([i1, i2], axis=1)
    topk_weight = jnp.stack([w1, w2], axis=1)
    return topk_idx, topk_weight
